# Initial kernel scaffold; baseline (speedup 1.0000x reference)
import jax
import jax.numpy as jnp
from jax.experimental import pallas as pl


def kernel(x, edge_index, params):
    raise NotImplementedError("write your pallas kernel here")


# speedup vs baseline: 2.1646x; 2.1646x over previous
import functools

import jax
import jax.numpy as jnp
from jax import lax
from jax.experimental import pallas as pl
from jax.experimental.pallas import tpu as pltpu
from jax.experimental.pallas import tpu_sc as plsc

N = 10000
E = 320000
D = 128
NC = 40

NCORES = 2
NSUB = 16
W = NCORES * NSUB
PER = E // W
C = 128
K = 80
KX = 82
PAD = KX * C - PER

NP = 10112
RPT = NP // NSUB

_mesh = plsc.VectorSubcoreMesh(
    core_axis_name="c", subcore_axis_name="s", num_cores=NCORES,
    num_subcores=NSUB)


_AGG_OUT = jax.ShapeDtypeStruct((NCORES, NP, D), jnp.float32)
_AGG_SCRATCH = [
    pltpu.VMEM((2, C), jnp.int32),
    pltpu.VMEM((2, C), jnp.int32),
    pltpu.VMEM((2, C, D), jnp.float32),
    pltpu.VMEM_SHARED((NP, D), jnp.float32),
    [pltpu.SemaphoreType.DMA] * 2,
    [pltpu.SemaphoreType.DMA] * 2,
]


def _sc_agg_body(h_hbm, srcw_hbm, dstw_hbm, zeros_hbm, out_hbm,
                 src_v, dst_v, rows_v, acc_sh, semi, semg):
    c = lax.axis_index("c")
    s = lax.axis_index("s")
    w = c * NSUB + s

    def start_idx(j, b):
        pltpu.async_copy(srcw_hbm.at[w, j], src_v.at[b], semi[b])
        pltpu.async_copy(dstw_hbm.at[w, j], dst_v.at[b], semi[b])

    def wait_idx(j, b):
        pltpu.make_async_copy(srcw_hbm.at[w, j], src_v.at[b], semi[b]).wait()
        pltpu.make_async_copy(dstw_hbm.at[w, j], dst_v.at[b], semi[b]).wait()

    def start_gather(b):
        pltpu.async_copy(h_hbm.at[src_v.at[b]], rows_v.at[b], semg[b])

    def wait_gather(b):
        pltpu.make_async_copy(h_hbm.at[src_v.at[b]], rows_v.at[b],
                              semg[b]).wait()

    start_idx(0, 0)
    start_idx(1, 1)
    pltpu.sync_copy(zeros_hbm, acc_sh.at[pl.ds(s * RPT, RPT)])
    plsc.subcore_barrier()
    wait_idx(0, 0)
    start_gather(0)

    def slot(j, b, bn):
        wait_gather(b)
        wait_idx(j + 1, bn)
        start_gather(bn)
        pltpu.sync_copy(rows_v.at[b], acc_sh.at[dst_v.at[b]], add=True)
        start_idx(j + 2, b)

    def body(i, carry):
        j0 = 2 * i
        slot(j0, 0, 1)
        slot(j0 + 1, 1, 0)
        return carry

    lax.fori_loop(0, K // 2, body, 0)
    wait_gather(0)
    wait_idx(K + 1, 1)
    plsc.subcore_barrier()
    pltpu.sync_copy(acc_sh.at[pl.ds(s * RPT, RPT)],
                    out_hbm.at[c, pl.ds(s * RPT, RPT)])


_sc_agg = pl.kernel(_sc_agg_body, out_type=_AGG_OUT, mesh=_mesh,
                    scratch_types=_AGG_SCRATCH)


def _mean_from_parts(parts_ref, cntp_ref):
    sums = parts_ref[0, :N, :] + parts_ref[1, :N, :]
    cnt = cntp_ref[0, :N, 0:1] + cntp_ref[1, :N, 0:1]
    inv = 1.0 / jnp.maximum(cnt, 1.0)
    return sums * inv


def _tc_layer_body(parts_ref, cntp_ref, h_ref, wl_ref, wr_ref, b_ref,
                   g_ref, be_ref, out_ref):
    mean = _mean_from_parts(parts_ref, cntp_ref)
    z = (jnp.dot(mean, wl_ref[...], preferred_element_type=jnp.float32)
         + jnp.dot(h_ref[...], wr_ref[...], preferred_element_type=jnp.float32)
         + b_ref[0])
    a = jnp.maximum(z, 0.0)
    mu = jnp.mean(a, axis=0, keepdims=True)
    var = jnp.mean((a - mu) ** 2, axis=0, keepdims=True)
    out_ref[...] = (a - mu) / jnp.sqrt(var + 1e-5) * g_ref[0] + be_ref[0]


def _tc_final_body(parts_ref, cntp_ref, h_ref, wl_ref, wr_ref, b_ref, out_ref):
    mean = _mean_from_parts(parts_ref, cntp_ref)
    z = (jnp.dot(mean, wl_ref[...], preferred_element_type=jnp.float32)
         + jnp.dot(h_ref[...], wr_ref[...], preferred_element_type=jnp.float32)
         + b_ref[0])
    m = jnp.max(z, axis=1, keepdims=True)
    lse = jnp.log(jnp.sum(jnp.exp(z - m), axis=1, keepdims=True))
    out_ref[...] = z - m - lse


_tc_layer = pl.pallas_call(
    _tc_layer_body,
    out_shape=jax.ShapeDtypeStruct((N, D), jnp.float32),
)

_tc_final = pl.pallas_call(
    _tc_final_body,
    out_shape=jax.ShapeDtypeStruct((N, NC), jnp.float32),
)


def kernel(x, edge_index, params):
    src = edge_index[0]
    dst = edge_index[1]
    srcw = jnp.pad(src.reshape(W, PER), ((0, 0), (0, PAD))).reshape(W, KX, C)
    dstw = jnp.pad(dst.reshape(W, PER), ((0, 0), (0, PAD)),
                   constant_values=N).reshape(W, KX, C)
    zeros_d = jnp.zeros((RPT, D), jnp.float32)

    cntp = _sc_agg(jnp.ones((N, D), jnp.float32), srcw, dstw, zeros_d)

    def sage_parts(h):
        return _sc_agg(h, srcw, dstw, zeros_d)

    def r2(v):
        return v.reshape(1, -1)

    p = params["proj"]
    h = _tc_layer(sage_parts(x), cntp, x, p["Wl"], p["Wr"], r2(p["b"]),
                  r2(params["norm0"]["g"]), r2(params["norm0"]["b"]))
    for p in params["layers"]:
        h = _tc_layer(sage_parts(h), cntp, h, p["Wl"], p["Wr"], r2(p["b"]),
                      r2(p["g"]), r2(p["be"]))
    p = params["final"]
    return _tc_final(sage_parts(h), cntp, h, p["Wl"], p["Wr"], r2(p["b"]))


# speedup vs baseline: 2.5229x; 1.1655x over previous
import functools

import jax
import jax.numpy as jnp
from jax import lax
from jax.experimental import pallas as pl
from jax.experimental.pallas import tpu as pltpu
from jax.experimental.pallas import tpu_sc as plsc

N = 10000
E = 320000
D = 128
NC = 40

NCORES = 2
NSUB = 16
W = NCORES * NSUB
PER = E // W
C = 128
K = 80
KX = 82
PAD = KX * C - PER

NP = 10112
RPT = NP // NSUB

_mesh = plsc.VectorSubcoreMesh(
    core_axis_name="c", subcore_axis_name="s", num_cores=NCORES,
    num_subcores=NSUB)


_AGG_OUT = jax.ShapeDtypeStruct((NCORES, NP, D), jnp.float32)
_AGG_SCRATCH = [
    pltpu.VMEM((2, 2, C), jnp.int32),
    pltpu.VMEM((2, C, D), jnp.float32),
    pltpu.VMEM_SHARED((NP, D), jnp.float32),
    [pltpu.SemaphoreType.DMA] * 2,
    [pltpu.SemaphoreType.DMA] * 2,
]


def _sc_agg_body(h_hbm, sdw_hbm, zeros_hbm, out_hbm,
                 idx_v, rows_v, acc_sh, semi, semg):
    c = lax.axis_index("c")
    s = lax.axis_index("s")
    w = c * NSUB + s

    def start_idx(j, b):
        pltpu.async_copy(sdw_hbm.at[w, j], idx_v.at[b], semi[b])

    def wait_idx(j, b):
        pltpu.make_async_copy(sdw_hbm.at[w, j], idx_v.at[b], semi[b]).wait()

    def start_gather(b):
        pltpu.async_copy(h_hbm.at[idx_v.at[b, 0]], rows_v.at[b], semg[b])

    def wait_gather(b):
        pltpu.make_async_copy(h_hbm.at[idx_v.at[b, 0]], rows_v.at[b],
                              semg[b]).wait()

    start_idx(0, 0)
    start_idx(1, 1)
    pltpu.sync_copy(zeros_hbm, acc_sh.at[pl.ds(s * RPT, RPT)])
    plsc.subcore_barrier()
    wait_idx(0, 0)
    start_gather(0)

    def slot(j, b, bn):
        wait_gather(b)
        wait_idx(j + 1, bn)
        start_gather(bn)
        pltpu.sync_copy(rows_v.at[b], acc_sh.at[idx_v.at[b, 1]], add=True)
        start_idx(j + 2, b)

    def body(i, carry):
        j0 = 2 * i
        slot(j0, 0, 1)
        slot(j0 + 1, 1, 0)
        return carry

    lax.fori_loop(0, K // 2, body, 0)
    wait_gather(0)
    wait_idx(K + 1, 1)
    plsc.subcore_barrier()
    pltpu.sync_copy(acc_sh.at[pl.ds(s * RPT, RPT)],
                    out_hbm.at[c, pl.ds(s * RPT, RPT)])


_sc_agg = pl.kernel(_sc_agg_body, out_type=_AGG_OUT, mesh=_mesh,
                    scratch_types=_AGG_SCRATCH)

_CNT_SCRATCH = [
    pltpu.VMEM((2, C), jnp.int32),
    pltpu.VMEM((C, D), jnp.float32),
    pltpu.VMEM_SHARED((NP, D), jnp.float32),
    [pltpu.SemaphoreType.DMA] * 2,
]


def _sc_count_body(dstw_hbm, ones_hbm, zeros_hbm, out_hbm,
                   dst_v, ones_v, cnt_sh, semi):
    c = lax.axis_index("c")
    s = lax.axis_index("s")
    w = c * NSUB + s

    def start_idx(j, b):
        pltpu.async_copy(dstw_hbm.at[w, j], dst_v.at[b], semi[b])

    def wait_idx(j, b):
        pltpu.make_async_copy(dstw_hbm.at[w, j], dst_v.at[b], semi[b]).wait()

    start_idx(0, 0)
    start_idx(1, 1)
    pltpu.sync_copy(ones_hbm, ones_v)
    pltpu.sync_copy(zeros_hbm, cnt_sh.at[pl.ds(s * RPT, RPT)])
    plsc.subcore_barrier()

    def slot(j, b):
        wait_idx(j, b)
        pltpu.sync_copy(ones_v, cnt_sh.at[dst_v.at[b]], add=True)
        start_idx(j + 2, b)

    def body(i, carry):
        j0 = 2 * i
        slot(j0, 0)
        slot(j0 + 1, 1)
        return carry

    lax.fori_loop(0, K // 2, body, 0)
    wait_idx(K, 0)
    wait_idx(K + 1, 1)
    plsc.subcore_barrier()
    pltpu.sync_copy(cnt_sh.at[pl.ds(s * RPT, RPT)],
                    out_hbm.at[c, pl.ds(s * RPT, RPT)])


_sc_count = pl.kernel(_sc_count_body, out_type=_AGG_OUT, mesh=_mesh,
                      scratch_types=_CNT_SCRATCH)


def _mean_from_parts(parts_ref, cntp_ref):
    sums = parts_ref[0, :N, :] + parts_ref[1, :N, :]
    cnt = cntp_ref[0, :N, 0:1] + cntp_ref[1, :N, 0:1]
    inv = 1.0 / jnp.maximum(cnt, 1.0)
    return sums * inv


def _tc_layer_body(parts_ref, cntp_ref, h_ref, wl_ref, wr_ref, b_ref,
                   g_ref, be_ref, out_ref):
    mean = _mean_from_parts(parts_ref, cntp_ref)
    z = (jnp.dot(mean, wl_ref[...], preferred_element_type=jnp.float32)
         + jnp.dot(h_ref[...], wr_ref[...], preferred_element_type=jnp.float32)
         + b_ref[0])
    a = jnp.maximum(z, 0.0)
    mu = jnp.mean(a, axis=0, keepdims=True)
    var = jnp.mean((a - mu) ** 2, axis=0, keepdims=True)
    out_ref[...] = (a - mu) / jnp.sqrt(var + 1e-5) * g_ref[0] + be_ref[0]


def _tc_final_body(parts_ref, cntp_ref, h_ref, wl_ref, wr_ref, b_ref, out_ref):
    mean = _mean_from_parts(parts_ref, cntp_ref)
    z = (jnp.dot(mean, wl_ref[...], preferred_element_type=jnp.float32)
         + jnp.dot(h_ref[...], wr_ref[...], preferred_element_type=jnp.float32)
         + b_ref[0])
    m = jnp.max(z, axis=1, keepdims=True)
    lse = jnp.log(jnp.sum(jnp.exp(z - m), axis=1, keepdims=True))
    out_ref[...] = z - m - lse


_tc_layer = pl.pallas_call(
    _tc_layer_body,
    out_shape=jax.ShapeDtypeStruct((N, D), jnp.float32),
)

_tc_final = pl.pallas_call(
    _tc_final_body,
    out_shape=jax.ShapeDtypeStruct((N, NC), jnp.float32),
)


def kernel(x, edge_index, params):
    src = edge_index[0]
    dst = edge_index[1]
    srcw = jnp.pad(src.reshape(W, PER), ((0, 0), (0, PAD))).reshape(W, KX, C)
    dstw = jnp.pad(dst.reshape(W, PER), ((0, 0), (0, PAD)),
                   constant_values=N).reshape(W, KX, C)
    sdw = jnp.stack([srcw, dstw], axis=2)
    zeros_d = jnp.zeros((RPT, D), jnp.float32)

    cntp = _sc_count(dstw, jnp.ones((C, D), jnp.float32), zeros_d)

    def sage_parts(h):
        return _sc_agg(h, sdw, zeros_d)

    def r2(v):
        return v.reshape(1, -1)

    p = params["proj"]
    h = _tc_layer(sage_parts(x), cntp, x, p["Wl"], p["Wr"], r2(p["b"]),
                  r2(params["norm0"]["g"]), r2(params["norm0"]["b"]))
    for p in params["layers"]:
        h = _tc_layer(sage_parts(h), cntp, h, p["Wl"], p["Wr"], r2(p["b"]),
                      r2(p["g"]), r2(p["be"]))
    p = params["final"]
    return _tc_final(sage_parts(h), cntp, h, p["Wl"], p["Wr"], r2(p["b"]))
